# speedup vs baseline: 1.4597x; 1.4597x over previous
import functools

import jax
import jax.numpy as jnp
from jax import lax
from jax.experimental import pallas as pl
from jax.experimental.pallas import tpu as pltpu
from jax.experimental.pallas import tpu_sc as plsc

_NC, _NS = 2, 16
_NW = _NC * _NS


@functools.lru_cache(maxsize=None)
def _build(B: int, D: int):
    b_per_w = B // _NW
    C = 64
    n_chunks = b_per_w // C
    mesh = plsc.VectorSubcoreMesh(core_axis_name="c", subcore_axis_name="s")

    @functools.partial(
        pl.kernel,
        out_type=jax.ShapeDtypeStruct((B, D), jnp.float32),
        mesh=mesh,
        scratch_types=[
            pltpu.VMEM((n_chunks, C), jnp.int32),
            pltpu.VMEM((C, D), jnp.float32),
            pltpu.SemaphoreType.DMA,
        ],
    )
    def gather_kernel(table_hbm, idx_hbm, out_hbm, idx_v, rows_v, sem):
        wid = lax.axis_index("s") * _NC + lax.axis_index("c")
        base = wid * b_per_w
        for j in range(n_chunks):
            pltpu.sync_copy(idx_hbm.at[pl.ds(base + j * C, C)], idx_v.at[j])
        for j in range(n_chunks):
            pltpu.async_copy(table_hbm.at[idx_v.at[j]], rows_v, sem).wait()
            pltpu.sync_copy(rows_v, out_hbm.at[pl.ds(base + j * C, C)])

    return gather_kernel


def kernel(pe, pos):
    B, = pos.shape
    D = pe.shape[1]
    return _build(B, D)(pe, pos.astype(jnp.int32))


# speedup vs baseline: 1.4798x; 1.0137x over previous
import functools

import jax
import jax.numpy as jnp
from jax import lax
from jax.experimental import pallas as pl
from jax.experimental.pallas import tpu as pltpu
from jax.experimental.pallas import tpu_sc as plsc

_NC, _NS = 2, 16
_NW = _NC * _NS


@functools.lru_cache(maxsize=None)
def _build(B: int, D: int):
    b_per_w = B // _NW
    C = 32
    n_chunks = b_per_w // C
    NBUF = 3
    mesh = plsc.VectorSubcoreMesh(core_axis_name="c", subcore_axis_name="s")

    @functools.partial(
        pl.kernel,
        out_type=jax.ShapeDtypeStruct((B, D), jnp.float32),
        mesh=mesh,
        scratch_types=[
            pltpu.VMEM((n_chunks, C), jnp.int32),
            pltpu.VMEM((NBUF, C, D), jnp.float32),
            pltpu.SemaphoreType.DMA((NBUF,)),
            pltpu.SemaphoreType.DMA((NBUF,)),
        ],
    )
    def gather_kernel(table_hbm, idx_hbm, out_hbm, idx_v, rows_v, gsem, wsem):
        wid = lax.axis_index("s") * _NC + lax.axis_index("c")
        base = wid * b_per_w
        for j in range(n_chunks):
            pltpu.sync_copy(idx_hbm.at[pl.ds(base + j * C, C)], idx_v.at[j])

        def gather_start(j):
            b = j % NBUF
            pltpu.async_copy(table_hbm.at[idx_v.at[j]], rows_v.at[b],
                             gsem.at[b])

        def write_start(j):
            b = j % NBUF
            pltpu.async_copy(rows_v.at[b],
                             out_hbm.at[pl.ds(base + j * C, C)], wsem.at[b])

        def gather_wait(j):
            b = j % NBUF
            pltpu.make_async_copy(table_hbm.at[idx_v.at[j]], rows_v.at[b],
                                  gsem.at[b]).wait()

        def write_wait(j):
            b = j % NBUF
            pltpu.make_async_copy(rows_v.at[b],
                                  out_hbm.at[pl.ds(base + j * C, C)],
                                  wsem.at[b]).wait()

        for j in range(min(NBUF, n_chunks)):
            gather_start(j)
        for j in range(n_chunks):
            gather_wait(j)
            write_start(j)
            nxt = j + NBUF
            if nxt < n_chunks:
                write_wait(nxt - NBUF)
                gather_start(nxt)
        for j in range(max(0, n_chunks - NBUF), n_chunks):
            write_wait(j)

    return gather_kernel


def kernel(pe, pos):
    B, = pos.shape
    D = pe.shape[1]
    return _build(B, D)(pe, pos.astype(jnp.int32))


# speedup vs baseline: 1.5626x; 1.0560x over previous
import functools

import jax
import jax.numpy as jnp
from jax import lax
from jax.experimental import pallas as pl
from jax.experimental.pallas import tpu as pltpu
from jax.experimental.pallas import tpu_sc as plsc

_NC, _NS = 2, 16
_NW = _NC * _NS


@functools.lru_cache(maxsize=None)
def _build(B: int, D: int):
    b_per_w = B // _NW
    C = 32
    n_chunks = b_per_w // C
    NBUF = 3
    mesh = plsc.VectorSubcoreMesh(core_axis_name="c", subcore_axis_name="s")

    @functools.partial(
        pl.kernel,
        out_type=jax.ShapeDtypeStruct((B, D), jnp.float32),
        mesh=mesh,
        scratch_types=[
            pltpu.VMEM((b_per_w,), jnp.int32),
            pltpu.VMEM((NBUF, C, D), jnp.float32),
            pltpu.SemaphoreType.DMA((NBUF,)),
            pltpu.SemaphoreType.DMA((NBUF,)),
        ],
    )
    def gather_kernel(table_hbm, idx_hbm, out_hbm, idx_v, rows_v, gsem, wsem):
        wid = lax.axis_index("s") * _NC + lax.axis_index("c")
        base = wid * b_per_w
        pltpu.sync_copy(idx_hbm.at[pl.ds(base, b_per_w)], idx_v)

        def gather_start(j):
            b = j % NBUF
            pltpu.async_copy(table_hbm.at[idx_v.at[pl.ds(j * C, C)]],
                             rows_v.at[b], gsem.at[b])

        def write_start(j):
            b = j % NBUF
            pltpu.async_copy(rows_v.at[b],
                             out_hbm.at[pl.ds(base + j * C, C)], wsem.at[b])

        def gather_wait(j):
            b = j % NBUF
            pltpu.make_async_copy(table_hbm.at[idx_v.at[pl.ds(j * C, C)]],
                                  rows_v.at[b], gsem.at[b]).wait()

        def write_wait(j):
            b = j % NBUF
            pltpu.make_async_copy(rows_v.at[b],
                                  out_hbm.at[pl.ds(base + j * C, C)],
                                  wsem.at[b]).wait()

        for j in range(min(NBUF, n_chunks)):
            gather_start(j)
        for j in range(n_chunks):
            gather_wait(j)
            write_start(j)
            nxt = j + NBUF
            if nxt < n_chunks:
                write_wait(nxt - NBUF)
                gather_start(nxt)
        for j in range(max(0, n_chunks - NBUF), n_chunks):
            write_wait(j)

    return gather_kernel


def kernel(pe, pos):
    B, = pos.shape
    D = pe.shape[1]
    return _build(B, D)(pe, pos.astype(jnp.int32))
